# Initial kernel scaffold; baseline (speedup 1.0000x reference)
import jax
import jax.numpy as jnp
from jax.experimental import pallas as pl


def kernel(useruin, gender, region_code, language, platform, device, age, grade, city_level, user_table, gender_table, region_table, language_table, platform_table, device_table, age_table, grade_table, city_level_table, W1, b1, W2, b2):
    raise NotImplementedError("write your pallas kernel here")


# speedup vs baseline: 1.2176x; 1.2176x over previous
import functools

import jax
import jax.numpy as jnp
import numpy as np
from jax import lax
from jax.experimental import pallas as pl
from jax.experimental.pallas import tpu as pltpu
from jax.experimental.pallas import tpu_sc as plsc

_B = 16384
_DU = 64
_UV = 1000000
_SMALL_DIMS = (8, 16, 8, 8, 16, 8, 8, 8)
_SMALL_VOCAB = (4, 1000, 100, 10, 1000, 100, 10, 10)
_SP = 128
_H = 256
_DM = 128

_FLAT_OFF = tuple(int(o) for o in np.cumsum(
    [0] + [v * d for v, d in zip(_SMALL_VOCAB, _SMALL_DIMS)]))
_FLAT_LEN = _FLAT_OFF[-1]

_info = plsc.get_sparse_core_info()
_NC, _NS = _info.num_cores, _info.num_subcores
_NW = _NC * _NS
_BPW = _B // _NW
_HALF = _BPW // 2
_GRP = _HALF // 16

_sc_mesh = plsc.VectorSubcoreMesh(core_axis_name="c", subcore_axis_name="s")


@functools.partial(
    pl.kernel,
    mesh=_sc_mesh,
    out_type=(
        jax.ShapeDtypeStruct((_B, 2 * _DU), jnp.float32),
        jax.ShapeDtypeStruct((_B, _SP), jnp.float32),
    ),
    scratch_types=(
        [pltpu.VMEM((_BPW,), jnp.int32) for _ in range(9)]
        + [
            pltpu.VMEM((_BPW,), jnp.int32),
            pltpu.VMEM((_FLAT_LEN,), jnp.float32),
            pltpu.VMEM((_HALF, 2 * _DU), jnp.float32),
            pltpu.VMEM((_HALF, _SP), jnp.float32),
            pltpu.SemaphoreType.DMA,
        ]
    ),
    compiler_params=pltpu.CompilerParams(needs_layout_passes=False),
)
def _sc_gather(*refs):
    idx_hbm = refs[0:9]
    user_pairs = refs[9]
    small_flat = refs[10]
    user_out = refs[11]
    small_out = refs[12]
    idx_v = refs[13:22]
    pair_v = refs[22]
    tbl_v = refs[23]
    user_v = refs[24]
    small_v = refs[25]
    sem = refs[26]

    wid = lax.axis_index("s") * _NC + lax.axis_index("c")
    base = wid * _BPW

    pltpu.sync_copy(small_flat, tbl_v)
    for i in range(9):
        pltpu.sync_copy(idx_hbm[i].at[pl.ds(base, _BPW)], idx_v[i])

    for k in range(_BPW // 16):
        s = pl.ds(k * 16, 16)
        pair_v[s] = jax.lax.shift_right_logical(idx_v[0][s], 1)

    lane = lax.iota(jnp.int32, 16)

    for h in range(2):
        hb = h * _HALF
        copies = [
            pltpu.async_copy(
                user_pairs.at[pair_v.at[pl.ds(hb + c * 128, 128)]],
                user_v.at[pl.ds(c * 128, 128)],
                sem,
            )
            for c in range(2)
        ]

        def body(g, carry):
            rows = g * 16 + lane
            for t in range(8):
                d = _SMALL_DIMS[t]
                fo = _FLAT_OFF[t]
                idx16 = idx_v[t + 1][pl.ds(hb + g * 16, 16)]
                addr = idx16 * d + fo
                for j in range(d):
                    vals = plsc.load_gather(tbl_v, [addr + j])
                    plsc.store_scatter(
                        small_v,
                        [rows, jnp.full((16,), 16 * t + j, jnp.int32)],
                        vals,
                    )
            return carry

        lax.fori_loop(0, _GRP, body, 0)

        for c in copies:
            c.wait()
        pltpu.sync_copy(user_v, user_out.at[pl.ds(base + hb, _HALF)])
        pltpu.sync_copy(small_v, small_out.at[pl.ds(base + hb, _HALF)])


_BM = 1024


def _ffn_body(par_ref, up_ref, xs_ref, w1p_ref, w1s_ref, b1_ref, w2_ref,
              b2_ref, o_ref):
    par = jnp.bitwise_and(par_ref[...], 1)
    li = lax.broadcasted_iota(jnp.int32, (_BM, 2 * _DU), 1)
    keep = jnp.right_shift(li, 6) == par
    xu = jnp.where(keep, up_ref[...], 0.0)
    acc = (
        jnp.dot(xu, w1p_ref[...], preferred_element_type=jnp.float32)
        + jnp.dot(xs_ref[...], w1s_ref[...],
                  preferred_element_type=jnp.float32)
        + b1_ref[...]
    )
    h = jnp.maximum(acc, 0.0)
    o_ref[...] = (
        jnp.dot(h, w2_ref[...], preferred_element_type=jnp.float32)
        + b2_ref[...]
    )


_ffn = pl.pallas_call(
    _ffn_body,
    grid=(_B // _BM,),
    in_specs=[
        pl.BlockSpec((_BM, 1), lambda i: (i, 0)),
        pl.BlockSpec((_BM, 2 * _DU), lambda i: (i, 0)),
        pl.BlockSpec((_BM, _SP), lambda i: (i, 0)),
        pl.BlockSpec((2 * _DU, _H), lambda i: (0, 0)),
        pl.BlockSpec((_SP, _H), lambda i: (0, 0)),
        pl.BlockSpec((1, _H), lambda i: (0, 0)),
        pl.BlockSpec((_H, _DM), lambda i: (0, 0)),
        pl.BlockSpec((1, _DM), lambda i: (0, 0)),
    ],
    out_specs=pl.BlockSpec((_BM, _DM), lambda i: (i, 0)),
    out_shape=jax.ShapeDtypeStruct((_B, _DM), jnp.float32),
)

_PACK_ROWS = np.concatenate(
    [16 * t + np.arange(d) for t, d in enumerate(_SMALL_DIMS)])


def kernel(useruin, gender, region_code, language, platform, device, age,
           grade, city_level, user_table, gender_table, region_table,
           language_table, platform_table, device_table, age_table,
           grade_table, city_level_table, W1, b1, W2, b2):
    idxs = [
        x.astype(jnp.int32)
        for x in (useruin, gender, region_code, language, platform, device,
                  age, grade, city_level)
    ]
    small_flat = jnp.concatenate([
        t.reshape(-1)
        for t in (gender_table, region_table, language_table, platform_table,
                  device_table, age_table, grade_table, city_level_table)
    ])
    user_pairs_tbl = user_table.reshape(_UV // 2, 2 * _DU)
    user_pairs, small_emb = _sc_gather(*idxs, user_pairs_tbl, small_flat)

    w1p = jnp.concatenate([W1[:_DU], W1[:_DU]], axis=0)
    w1s = jnp.zeros((_SP, _H), jnp.float32).at[_PACK_ROWS].set(W1[_DU:])
    return _ffn(idxs[0].reshape(_B, 1), user_pairs, small_emb, w1p, w1s,
                b1.reshape(1, _H), W2, b2.reshape(1, _DM))


# speedup vs baseline: 1.3147x; 1.0798x over previous
import functools

import jax
import jax.numpy as jnp
import numpy as np
from jax import lax
from jax.experimental import pallas as pl
from jax.experimental.pallas import tpu as pltpu
from jax.experimental.pallas import tpu_sc as plsc

_B = 16384
_DU = 64
_UV = 1000000
_SMALL_DIMS = (8, 16, 8, 8, 16, 8, 8, 8)
_SMALL_VOCAB = (4, 1000, 100, 10, 1000, 100, 10, 10)
_SP = 128
_H = 256
_DM = 128

_FLAT_OFF = tuple(int(o) for o in np.cumsum(
    [0] + [v * d for v, d in zip(_SMALL_VOCAB, _SMALL_DIMS)]))
_FLAT_LEN = _FLAT_OFF[-1]

_info = plsc.get_sparse_core_info()
_NC, _NS = _info.num_cores, _info.num_subcores
_NW = _NC * _NS
_BPW = _B // _NW
_GRP = _BPW // 16

_sc_mesh = plsc.VectorSubcoreMesh(core_axis_name="c", subcore_axis_name="s")


@functools.partial(
    pl.kernel,
    mesh=_sc_mesh,
    out_type=(
        jax.ShapeDtypeStruct((_B, _DU), jnp.float32),
        jax.ShapeDtypeStruct((_B, _SP), jnp.float32),
    ),
    scratch_types=(
        [pltpu.VMEM((_BPW,), jnp.int32) for _ in range(8)]
        + [
            pltpu.VMEM((_BPW,), jnp.int32),
            pltpu.VMEM((_FLAT_LEN,), jnp.float32),
            pltpu.VMEM((_BPW, _SP), jnp.float32),
            pltpu.SemaphoreType.DMA,
            pltpu.SemaphoreType.DMA,
        ]
    ),
    compiler_params=pltpu.CompilerParams(needs_layout_passes=False),
)
def _sc_gather(*refs):
    idx_hbm = refs[0:9]
    user_tbl = refs[9]
    small_flat = refs[10]
    user_out = refs[11]
    small_out = refs[12]
    idx_v = refs[13:21]
    uidx_v = refs[21]
    tbl_v = refs[22]
    small_v = refs[23]
    sem = refs[24]
    usem = refs[25]

    wid = lax.axis_index("s") * _NC + lax.axis_index("c")
    base = wid * _BPW

    pltpu.sync_copy(idx_hbm[0].at[pl.ds(base, _BPW)], uidx_v)
    pltpu.sync_copy(small_flat, tbl_v)
    for i in range(8):
        pltpu.sync_copy(idx_hbm[i + 1].at[pl.ds(base, _BPW)], idx_v[i])

    lane = lax.iota(jnp.int32, 16)

    def body(g, carry):
        rows = g * 16 + lane
        uidx16 = uidx_v[pl.ds(g * 16, 16)]
        for j in range(16):
            s = jnp.max(jnp.where(lane == j, uidx16, 0))
            pltpu.async_copy(
                user_tbl.at[pl.ds(s, 1)],
                user_out.at[pl.ds(base + g * 16 + j, 1)],
                usem,
            )
        for t in range(8):
            d = _SMALL_DIMS[t]
            fo = _FLAT_OFF[t]
            idx16 = idx_v[t][pl.ds(g * 16, 16)]
            addr = idx16 * d + fo
            for j in range(d):
                vals = plsc.load_gather(tbl_v, [addr + j])
                plsc.store_scatter(
                    small_v,
                    [rows, jnp.full((16,), 16 * t + j, jnp.int32)],
                    vals,
                )
        return carry

    lax.fori_loop(0, _GRP, body, 0)

    pltpu.sync_copy(small_v, small_out.at[pl.ds(base, _BPW)])

    pltpu.make_async_copy(
        user_tbl.at[pl.ds(0, _BPW)],
        user_out.at[pl.ds(base, _BPW)],
        usem,
    ).wait()


_BM = 1024


def _ffn_body(xu_ref, xs_ref, w1u_ref, w1s_ref, b1_ref, w2_ref, b2_ref,
              o_ref):
    acc = (
        jnp.dot(xu_ref[...], w1u_ref[...], preferred_element_type=jnp.float32)
        + jnp.dot(xs_ref[...], w1s_ref[...],
                  preferred_element_type=jnp.float32)
        + b1_ref[...]
    )
    h = jnp.maximum(acc, 0.0)
    o_ref[...] = (
        jnp.dot(h, w2_ref[...], preferred_element_type=jnp.float32)
        + b2_ref[...]
    )


_ffn = pl.pallas_call(
    _ffn_body,
    grid=(_B // _BM,),
    in_specs=[
        pl.BlockSpec((_BM, _DU), lambda i: (i, 0)),
        pl.BlockSpec((_BM, _SP), lambda i: (i, 0)),
        pl.BlockSpec((_DU, _H), lambda i: (0, 0)),
        pl.BlockSpec((_SP, _H), lambda i: (0, 0)),
        pl.BlockSpec((1, _H), lambda i: (0, 0)),
        pl.BlockSpec((_H, _DM), lambda i: (0, 0)),
        pl.BlockSpec((1, _DM), lambda i: (0, 0)),
    ],
    out_specs=pl.BlockSpec((_BM, _DM), lambda i: (i, 0)),
    out_shape=jax.ShapeDtypeStruct((_B, _DM), jnp.float32),
)

_PACK_ROWS = np.concatenate(
    [16 * t + np.arange(d) for t, d in enumerate(_SMALL_DIMS)])


def kernel(useruin, gender, region_code, language, platform, device, age,
           grade, city_level, user_table, gender_table, region_table,
           language_table, platform_table, device_table, age_table,
           grade_table, city_level_table, W1, b1, W2, b2):
    idxs = [
        x.astype(jnp.int32)
        for x in (useruin, gender, region_code, language, platform, device,
                  age, grade, city_level)
    ]
    small_flat = jnp.concatenate([
        t.reshape(-1)
        for t in (gender_table, region_table, language_table, platform_table,
                  device_table, age_table, grade_table, city_level_table)
    ])
    user_emb, small_emb = _sc_gather(*idxs, user_table, small_flat)

    w1s = jnp.zeros((_SP, _H), jnp.float32).at[_PACK_ROWS].set(W1[_DU:])
    return _ffn(user_emb, small_emb, W1[:_DU], w1s,
                b1.reshape(1, _H), W2, b2.reshape(1, _DM))
